# baseline (speedup 1.0000x reference)
import functools

import jax
import jax.numpy as jnp
from jax.experimental import pallas as pl
from jax.experimental.pallas import tpu as pltpu

N_GRAPHS = 512


def _leaky(x, s=0.01):
    return jnp.where(x > 0, x, s * x)


def _elu(x):
    return jnp.where(x > 0, x, jnp.expm1(x))


def _segment_softmax(alpha, index, num_segments):
    amax = jax.ops.segment_max(alpha, index, num_segments)
    amax = jnp.where(jnp.isfinite(amax), amax, 0.0)
    a = jnp.exp(alpha - amax[index])
    denom = jax.ops.segment_sum(a, index, num_segments)
    return a / (denom[index] + 1e-16)


def _gru(x, h, wih, whh, bih, bhh):
    gi = x @ wih.T + bih
    gh = h @ whh.T + bhh
    ir, iz, inn = jnp.split(gi, 3, axis=-1)
    hr, hz, hn = jnp.split(gh, 3, axis=-1)
    r = jax.nn.sigmoid(ir + hr)
    z = jax.nn.sigmoid(iz + hz)
    n = jnp.tanh(inn + r * hn)
    return (1.0 - z) * n + z * h


def _gate_conv(x, edge_index, edge_attr, p, n_nodes):
    src, dst = edge_index[0], edge_index[1]
    xj = x[src]
    xi = x[dst]
    t = _leaky(jnp.concatenate([xj, edge_attr], axis=-1) @ p['gate_lin1_w'].T)
    alpha = _leaky(t @ p['gate_att_l'] + xi @ p['gate_att_r'])
    alpha = _segment_softmax(alpha, dst, n_nodes)
    msg = (xj @ p['gate_lin2_w'].T) * alpha[:, None]
    return jax.ops.segment_sum(msg, dst, n_nodes) + p['gate_bias']


def _gat_conv(x, edge_index, w, att_src, att_dst, bias, n_nodes):
    src, dst = edge_index[0], edge_index[1]
    xp = x @ w.T
    a = _leaky(xp[src] @ att_src + xp[dst] @ att_dst, 0.01)
    alpha = _segment_softmax(a, dst, n_nodes)
    out = jax.ops.segment_sum(alpha[:, None] * xp[src], dst, n_nodes)
    return out + bias


_BLK = 1024


def _matmul_kernel(x_ref, w_ref, b_ref, o_ref):
    o_ref[...] = jnp.dot(x_ref[...], w_ref[...],
                         preferred_element_type=jnp.float32) + b_ref[...]


def _pallas_linear(x, w_t, b):
    n, k = x.shape
    m = w_t.shape[1]
    npad = (-n) % _BLK
    xp = jnp.pad(x, ((0, npad), (0, 0)))
    grid = (xp.shape[0] // _BLK,)
    out = pl.pallas_call(
        _matmul_kernel,
        grid=grid,
        in_specs=[
            pl.BlockSpec((_BLK, k), lambda i: (i, 0)),
            pl.BlockSpec((k, m), lambda i: (0, 0)),
            pl.BlockSpec((1, m), lambda i: (0, 0)),
        ],
        out_specs=pl.BlockSpec((_BLK, m), lambda i: (i, 0)),
        out_shape=jax.ShapeDtypeStruct((xp.shape[0], m), jnp.float32),
    )(xp, w_t, b.reshape(1, m))
    return out[:n]


def _forward(x, edge_attr, params, edge_index, batch):
    n = x.shape[0]
    p = params
    has_edges = jnp.zeros((N_GRAPHS,), dtype=bool).at[batch[edge_index[0]]].set(True)
    x0 = _leaky(_pallas_linear(x, p['lin1_w'].T, p['lin1_b']))
    h = _elu(_gate_conv(x0, edge_index, edge_attr, p, n))
    xg = jax.nn.relu(_gru(h, x0, p['gru0_wih'], p['gru0_whh'], p['gru0_bih'], p['gru0_bhh']))
    for i in range(2):
        h = _elu(_gat_conv(xg, edge_index, p['conv%d_w' % i], p['conv%d_att_src' % i],
                           p['conv%d_att_dst' % i], p['conv%d_bias' % i], n))
        xg = jax.nn.relu(_gru(h, xg, p['agru%d_wih' % i], p['agru%d_whh' % i],
                              p['agru%d_bih' % i], p['agru%d_bhh' % i]))
    out = jax.nn.relu(jax.ops.segment_sum(xg, batch, N_GRAPHS))
    xs = xg @ p['mol_w'].T
    a_src = xs @ p['mol_att_src']
    for _ in range(2):
        xd = out @ p['mol_w'].T
        a = _leaky(a_src + (xd @ p['mol_att_dst'])[batch], 0.01)
        alpha = _segment_softmax(a, batch, N_GRAPHS)
        h = _elu(jax.ops.segment_sum(alpha[:, None] * xs, batch, N_GRAPHS) + p['mol_bias'])
        out = jax.nn.relu(_gru(h, out, p['mgru_wih'], p['mgru_whh'], p['mgru_bih'], p['mgru_bhh']))
    gnn_out = out
    xl = x0 @ p['linlone_w'].T + p['linlone_b']
    atom_out = jax.ops.segment_sum(xl, batch, N_GRAPHS)
    out = jnp.where(has_edges[:, None], gnn_out, atom_out)
    return _pallas_linear(out, p['lin2_w'].T, p['lin2_b'])


def kernel(x, edge_attr, params, edge_index, batch):
    return _forward(x, edge_attr, params, edge_index, batch)


# speedup vs baseline: 5.5766x; 5.5766x over previous
import functools

import jax
import jax.numpy as jnp
from jax import lax
from jax.experimental import pallas as pl
from jax.experimental.pallas import tpu as pltpu
from jax.experimental.pallas import tpu_sc as plsc

N_GRAPHS = 512
N_NODES = 50000
N_EDGES = 800000

EPAD = 802816
ROWS = EPAD // 128
TPW = ROWS // 32
NPAD = 50176
DUMMY = 50000
HALFP = 12800
ACCQ = 101
GBLK = 1792
F32 = jnp.float32
I32 = jnp.int32

_MESH = plsc.VectorSubcoreMesh(core_axis_name="c", subcore_axis_name="s")
_SC_PARAMS = pltpu.CompilerParams(needs_layout_passes=False)


def _wid():
    return lax.axis_index("s") * 2 + lax.axis_index("c")


def _leaky(x, s=0.01):
    return jnp.where(x > 0, x, s * x)


def _elu(x):
    return jnp.where(x > 0, x, jnp.expm1(x))


def _gru(x, h, wih, whh, bih, bhh):
    gi = x @ wih.T + bih
    gh = h @ whh.T + bhh
    ir, iz, inn = jnp.split(gi, 3, axis=-1)
    hr, hz, hn = jnp.split(gh, 3, axis=-1)
    r = jax.nn.sigmoid(ir + hr)
    z = jax.nn.sigmoid(iz + hz)
    n = jnp.tanh(inn + r * hn)
    return (1.0 - z) * n + z * h


def _segment_softmax(alpha, index, num_segments):
    amax = jax.ops.segment_max(alpha, index, num_segments)
    amax = jnp.where(jnp.isfinite(amax), amax, 0.0)
    a = jnp.exp(alpha - amax[index])
    denom = jax.ops.segment_sum(a, index, num_segments)
    return a / (denom[index] + 1e-16)


def _make_sc_gather(d):
    @functools.partial(
        pl.kernel,
        out_type=jax.ShapeDtypeStruct((EPAD, d), F32),
        mesh=_MESH,
        compiler_params=_SC_PARAMS,
        scratch_types=[
            pltpu.VMEM((TPW, 128), I32),
            pltpu.VMEM((128, d), F32),
            pltpu.SemaphoreType.DMA,
        ],
    )
    def k(tab_hbm, idx_hbm, out_hbm, idx_v, rows_v, sem):
        w = _wid()
        pltpu.sync_copy(idx_hbm.at[w], idx_v)

        def body(j, carry):
            pltpu.async_copy(tab_hbm.at[idx_v.at[j]], rows_v, sem).wait()
            pltpu.sync_copy(rows_v, out_hbm.at[pl.ds((w * TPW + j) * 128, 128)])
            return carry

        lax.fori_loop(0, TPW, body, 0)

    return k


def _make_sc_alpha(use_base, use_src):
    scratch = [pltpu.VMEM((NPAD,), F32)]
    if use_src:
        scratch.append(pltpu.VMEM((NPAD,), F32))
        scratch.append(pltpu.VMEM((GBLK,), I32))
    if use_base:
        scratch.append(pltpu.VMEM((GBLK,), F32))
    scratch += [
        pltpu.VMEM((GBLK,), I32),
        pltpu.VMEM((GBLK,), F32),
        pltpu.VMEM((16,), F32),
    ]

    @functools.partial(
        pl.kernel,
        out_type=(
            jax.ShapeDtypeStruct((EPAD,), F32),
            jax.ShapeDtypeStruct((32, 1, 16), F32),
        ),
        mesh=_MESH,
        compiler_params=_SC_PARAMS,
        scratch_types=scratch,
    )
    def k(*refs):
        it = iter(refs)
        base_hbm = next(it) if use_base else None
        src_hbm = next(it) if use_src else None
        dst_hbm = next(it)
        t1_hbm = next(it) if use_src else None
        t2_hbm = next(it)
        a_hbm = next(it)
        mx_hbm = next(it)
        t2_v = next(it)
        t1_v = next(it) if use_src else None
        s_v = next(it) if use_src else None
        base_v = next(it) if use_base else None
        d_v = next(it)
        st_v = next(it)
        mx_v = next(it)

        w = _wid()
        ebase = w * TPW * 128
        pltpu.sync_copy(t2_hbm, t2_v)
        if use_src:
            pltpu.sync_copy(t1_hbm, t1_v)

        mx = jnp.full((16,), -3.0e38, F32)
        for b in range(TPW * 128 // GBLK):
            off = ebase + b * GBLK
            if use_base:
                pltpu.sync_copy(base_hbm.at[pl.ds(off, GBLK)], base_v)
            if use_src:
                pltpu.sync_copy(src_hbm.at[pl.ds(off, GBLK)], s_v)
            pltpu.sync_copy(dst_hbm.at[pl.ds(off, GBLK)], d_v)

            def body(j, m):
                sl = pl.ds(j * 16, 16)
                v = plsc.load_gather(t2_v, [d_v[sl]])
                if use_src:
                    v = v + plsc.load_gather(t1_v, [s_v[sl]])
                if use_base:
                    v = v + base_v[sl]
                v = jnp.where(v > 0, v, 0.01 * v)
                st_v[sl] = v
                return jnp.maximum(m, v)

            mx = lax.fori_loop(0, GBLK // 16, body, mx)
            pltpu.sync_copy(st_v, a_hbm.at[pl.ds(off, GBLK)])
        mx_v[...] = mx
        pltpu.sync_copy(mx_v, mx_hbm.at[w, 0])

    return k


def _make_sc_exp_hist():
    @functools.partial(
        pl.kernel,
        out_type=(
            jax.ShapeDtypeStruct((EPAD,), F32),
            jax.ShapeDtypeStruct((32, 1, NPAD), F32),
        ),
        mesh=_MESH,
        compiler_params=_SC_PARAMS,
        scratch_types=[
            pltpu.VMEM((NPAD,), F32),
            pltpu.VMEM((GBLK,), F32),
            pltpu.VMEM((GBLK,), I32),
            pltpu.VMEM((GBLK,), F32),
            pltpu.VMEM((32, 1, 16), F32),
        ],
    )
    def k(a_hbm, dst_hbm, mx_hbm, e_hbm, part_hbm, den_v, a_v, d_v, e_v, tm_v):
        w = _wid()
        ebase = w * TPW * 128
        pltpu.sync_copy(mx_hbm, tm_v)
        m = tm_v[0, 0]
        for i in range(1, 32):
            m = jnp.maximum(m, tm_v[i, 0])
        g = jnp.max(m)

        def zbody(i, c):
            den_v[pl.ds(i * 16, 16)] = jnp.zeros((16,), F32)
            return c

        lax.fori_loop(0, NPAD // 16, zbody, 0)

        for b in range(TPW * 128 // GBLK):
            off = ebase + b * GBLK
            pltpu.sync_copy(a_hbm.at[pl.ds(off, GBLK)], a_v)
            pltpu.sync_copy(dst_hbm.at[pl.ds(off, GBLK)], d_v)

            def body(j, c):
                sl = pl.ds(j * 16, 16)
                e16 = jnp.exp(a_v[sl] - g)
                e_v[sl] = e16
                plsc.addupdate_scatter(den_v, [d_v[sl]], e16)
                return c

            lax.fori_loop(0, GBLK // 16, body, 0)
            pltpu.sync_copy(e_v, e_hbm.at[pl.ds(off, GBLK)])
        pltpu.sync_copy(den_v, part_hbm.at[w, 0])

    return k


def _make_sc_w():
    @functools.partial(
        pl.kernel,
        out_type=jax.ShapeDtypeStruct((EPAD,), F32),
        mesh=_MESH,
        compiler_params=_SC_PARAMS,
        scratch_types=[
            pltpu.VMEM((NPAD,), F32),
            pltpu.VMEM((GBLK,), F32),
            pltpu.VMEM((GBLK,), I32),
            pltpu.VMEM((GBLK,), F32),
        ],
    )
    def k(e_hbm, dst_hbm, den_hbm, w_hbm, den_v, e_v, d_v, w_v):
        w = _wid()
        ebase = w * TPW * 128
        pltpu.sync_copy(den_hbm, den_v)
        for b in range(TPW * 128 // GBLK):
            off = ebase + b * GBLK
            pltpu.sync_copy(e_hbm.at[pl.ds(off, GBLK)], e_v)
            pltpu.sync_copy(dst_hbm.at[pl.ds(off, GBLK)], d_v)

            def body(j, c):
                sl = pl.ds(j * 16, 16)
                den16 = plsc.load_gather(den_v, [d_v[sl]])
                w_v[sl] = e_v[sl] / (den16 + 1e-16)
                return c

            lax.fori_loop(0, GBLK // 16, body, 0)
            pltpu.sync_copy(w_v, w_hbm.at[pl.ds(off, GBLK)])

    return k


def _make_sc_deg():
    @functools.partial(
        pl.kernel,
        out_type=jax.ShapeDtypeStruct((32, 1, NPAD), F32),
        mesh=_MESH,
        compiler_params=_SC_PARAMS,
        scratch_types=[
            pltpu.VMEM((NPAD,), F32),
            pltpu.VMEM((GBLK,), I32),
        ],
    )
    def k(src_hbm, part_hbm, deg_v, s_v):
        w = _wid()
        ebase = w * TPW * 128

        def zbody(i, c):
            deg_v[pl.ds(i * 16, 16)] = jnp.zeros((16,), F32)
            return c

        lax.fori_loop(0, NPAD // 16, zbody, 0)
        ones = jnp.full((16,), 1.0, F32)
        for b in range(TPW * 128 // GBLK):
            off = ebase + b * GBLK
            pltpu.sync_copy(src_hbm.at[pl.ds(off, GBLK)], s_v)

            def body(j, c):
                plsc.addupdate_scatter(deg_v, [s_v[pl.ds(j * 16, 16)]], ones)
                return c

            lax.fori_loop(0, GBLK // 16, body, 0)
        pltpu.sync_copy(deg_v, part_hbm.at[w, 0])

    return k


def _make_sc_scatter():
    cpw = ROWS // 16

    @functools.partial(
        pl.kernel,
        out_type=jax.ShapeDtypeStruct((2 * HALFP, 128), F32),
        mesh=_MESH,
        compiler_params=_SC_PARAMS,
        scratch_types=[
            pltpu.VMEM_SHARED((ACCQ * 128, 128), F32),
            pltpu.VMEM((64, 128), F32),
            pltpu.VMEM((128,), I32),
            pltpu.VMEM((64,), I32),
            pltpu.VMEM((64,), I32),
        ],
    )
    def k(msg_hbm, dst_hbm, out_hbm, acc, rows_v, d_v, la_v, lb_v):
        c = lax.axis_index("c")
        s = lax.axis_index("s")

        def zv(i, cc):
            for q in range(8):
                rows_v[i, pl.ds(q * 16, 16)] = jnp.zeros((16,), F32)
            return cc

        lax.fori_loop(0, 64, zv, 0)

        def zacc(i, cc):
            ch = s + i * 16

            @pl.when(ch < 2 * ACCQ)
            def _():
                pltpu.sync_copy(rows_v, acc.at[pl.ds(ch * 64, 64)])

            return cc

        lax.fori_loop(0, (2 * ACCQ + 15) // 16, zacc, 0)
        plsc.subcore_barrier()

        pbase = c * HALFP

        def body(j, cc):
            chunk = s * cpw + j
            pltpu.sync_copy(dst_hbm.at[pl.ds(chunk * 128, 128)], d_v)
            for half, l_v in ((0, la_v), (1, lb_v)):
                for gi in range(4):
                    sl = pl.ds(half * 64 + gi * 16, 16)
                    loc = lax.shift_right_logical(d_v[sl], 1) - pbase
                    ok = jnp.logical_and(loc >= 0, loc < HALFP)
                    l_v[pl.ds(gi * 16, 16)] = jnp.where(ok, loc, HALFP)
            for half, l_v in ((0, la_v), (1, lb_v)):
                pltpu.sync_copy(
                    msg_hbm.at[pl.ds(chunk * 128 + half * 64, 64)], rows_v)
                pltpu.sync_copy(rows_v, acc.at[l_v], add=True)
            return cc

        lax.fori_loop(0, cpw, body, 0)
        plsc.subcore_barrier()
        rpw = HALFP // 16
        pltpu.sync_copy(
            acc.at[pl.ds(s * rpw, rpw)],
            out_hbm.at[pl.ds(c * HALFP + s * rpw, rpw)],
        )

    return k


_sc_gather128 = _make_sc_gather(128)
_sc_alpha_gat = _make_sc_alpha(use_base=False, use_src=True)
_sc_alpha_gate = _make_sc_alpha(use_base=True, use_src=False)
_sc_exp_hist = _make_sc_exp_hist()
_sc_w = _make_sc_w()
_sc_deg = _make_sc_deg()
_sc_scatter = _make_sc_scatter()


def _npad1(v):
    return jnp.pad(v, (0, NPAD - N_NODES))


def _npad2(m):
    return jnp.pad(m, ((0, NPAD - m.shape[0]), (0, 0)))


def _scatter_nodes(m, dstp):
    z = jnp.zeros_like(m)
    left = (dstp & 1) == 0
    msg = jnp.where(left[:, None], jnp.concatenate([m, z], axis=1),
                    jnp.concatenate([z, m], axis=1))
    out = _sc_scatter(msg, dstp).reshape(4 * HALFP, 64)
    return out[:N_NODES]


def _softmax_weights(alpha, tmax, dstp):
    e, parts = _sc_exp_hist(alpha, dstp, tmax)
    denom = parts.sum(axis=(0, 1))
    return _sc_w(e, dstp, denom)


_BLK = 1024


def _matmul_kernel(x_ref, w_ref, b_ref, o_ref):
    o_ref[...] = jnp.dot(x_ref[...], w_ref[...],
                         preferred_element_type=jnp.float32) + b_ref[...]


def _pallas_linear(x, w_t, b):
    n, kk = x.shape
    m = w_t.shape[1]
    npad = (-n) % _BLK
    xp = jnp.pad(x, ((0, npad), (0, 0)))
    grid = (xp.shape[0] // _BLK,)
    out = pl.pallas_call(
        _matmul_kernel,
        grid=grid,
        in_specs=[
            pl.BlockSpec((_BLK, kk), lambda i: (i, 0)),
            pl.BlockSpec((kk, m), lambda i: (0, 0)),
            pl.BlockSpec((1, m), lambda i: (0, 0)),
        ],
        out_specs=pl.BlockSpec((_BLK, m), lambda i: (i, 0)),
        out_shape=jax.ShapeDtypeStruct((xp.shape[0], m), jnp.float32),
    )(xp, w_t, b.reshape(1, m))
    return out[:n]


def _gate_conv_sc(x0, srcp, src2d, dstp, edge_attr, p):
    w1 = p['gate_lin1_w']
    u = x0 @ w1[:, :64].T
    y2 = x0 @ p['gate_lin2_w'].T
    rd = x0 @ p['gate_att_r']
    uy = _sc_gather128(_npad2(jnp.concatenate([u, y2], axis=1)), src2d)
    eproj = jnp.pad(edge_attr, ((0, EPAD - N_EDGES), (0, 0))) @ w1[:, 64:].T
    tdot = _leaky(uy[:, :64] + eproj) @ p['gate_att_l']
    alpha, tmax = _sc_alpha_gate(tdot, dstp, _npad1(rd))
    wts = _softmax_weights(alpha, tmax, dstp)
    return _scatter_nodes(uy[:, 64:] * wts[:, None], dstp) + p['gate_bias']


def _gat_conv_sc(xg, srcp, src2d, dstp, w, att_src, att_dst, bias):
    xp = xg @ w.T
    ssrc = xp @ att_src
    sdst = xp @ att_dst
    alpha, tmax = _sc_alpha_gat(srcp, dstp, _npad1(ssrc), _npad1(sdst))
    wts = _softmax_weights(alpha, tmax, dstp)
    xpj = _sc_gather128(_npad2(jnp.concatenate([xp, xp], axis=1)), src2d)
    return _scatter_nodes(xpj[:, 64:] * wts[:, None], dstp) + bias


def kernel(x, edge_attr, params, edge_index, batch):
    p = params
    src = edge_index[0].astype(I32)
    dst = edge_index[1].astype(I32)
    pad = jnp.full((EPAD - N_EDGES,), DUMMY, I32)
    srcp = jnp.concatenate([src, pad])
    dstp = jnp.concatenate([dst, pad])
    src2d = srcp.reshape(32, TPW, 128)

    deg_parts = _sc_deg(srcp)
    outdeg = deg_parts.sum(axis=(0, 1))[:N_NODES]
    has_edges = jax.ops.segment_sum(outdeg, batch, N_GRAPHS) > 0.5

    x0 = _leaky(_pallas_linear(x, p['lin1_w'].T, p['lin1_b']))
    h = _elu(_gate_conv_sc(x0, srcp, src2d, dstp, edge_attr, p))
    xg = jax.nn.relu(_gru(h, x0, p['gru0_wih'], p['gru0_whh'],
                          p['gru0_bih'], p['gru0_bhh']))
    for i in range(2):
        h = _elu(_gat_conv_sc(xg, srcp, src2d, dstp, p['conv%d_w' % i],
                              p['conv%d_att_src' % i], p['conv%d_att_dst' % i],
                              p['conv%d_bias' % i]))
        xg = jax.nn.relu(_gru(h, xg, p['agru%d_wih' % i], p['agru%d_whh' % i],
                              p['agru%d_bih' % i], p['agru%d_bhh' % i]))
    out = jax.nn.relu(jax.ops.segment_sum(xg, batch, N_GRAPHS))
    xs = xg @ p['mol_w'].T
    a_src = xs @ p['mol_att_src']
    for _ in range(2):
        xd = out @ p['mol_w'].T
        a = _leaky(a_src + (xd @ p['mol_att_dst'])[batch], 0.01)
        alpha = _segment_softmax(a, batch, N_GRAPHS)
        h = _elu(jax.ops.segment_sum(alpha[:, None] * xs, batch, N_GRAPHS)
                 + p['mol_bias'])
        out = jax.nn.relu(_gru(h, out, p['mgru_wih'], p['mgru_whh'],
                               p['mgru_bih'], p['mgru_bhh']))
    gnn_out = out
    xl = x0 @ p['linlone_w'].T + p['linlone_b']
    atom_out = jax.ops.segment_sum(xl, batch, N_GRAPHS)
    out = jnp.where(has_edges[:, None], gnn_out, atom_out)
    return _pallas_linear(out, p['lin2_w'].T, p['lin2_b'])


# speedup vs baseline: 5.8104x; 1.0419x over previous
import functools

import jax
import jax.numpy as jnp
from jax import lax
from jax.experimental import pallas as pl
from jax.experimental.pallas import tpu as pltpu
from jax.experimental.pallas import tpu_sc as plsc

N_GRAPHS = 512
N_NODES = 50000
N_EDGES = 800000

EPAD = 802816
ROWS = EPAD // 128
TPW = ROWS // 32
NPAD = 50176
DUMMY = 50000
HALFP = 12800
ACCQ = 101
GBLK = 1792
F32 = jnp.float32
I32 = jnp.int32

_MESH = plsc.VectorSubcoreMesh(core_axis_name="c", subcore_axis_name="s")
_SC_PARAMS = pltpu.CompilerParams(needs_layout_passes=False)


def _wid():
    return lax.axis_index("s") * 2 + lax.axis_index("c")


def _leaky(x, s=0.01):
    return jnp.where(x > 0, x, s * x)


def _elu(x):
    return jnp.where(x > 0, x, jnp.expm1(x))


def _gru(x, h, wih, whh, bih, bhh):
    gi = x @ wih.T + bih
    gh = h @ whh.T + bhh
    ir, iz, inn = jnp.split(gi, 3, axis=-1)
    hr, hz, hn = jnp.split(gh, 3, axis=-1)
    r = jax.nn.sigmoid(ir + hr)
    z = jax.nn.sigmoid(iz + hz)
    n = jnp.tanh(inn + r * hn)
    return (1.0 - z) * n + z * h


def _segment_softmax(alpha, index, num_segments):
    amax = jax.ops.segment_max(alpha, index, num_segments)
    amax = jnp.where(jnp.isfinite(amax), amax, 0.0)
    a = jnp.exp(alpha - amax[index])
    denom = jax.ops.segment_sum(a, index, num_segments)
    return a / (denom[index] + 1e-16)


def _make_sc_gather(d):
    nb = 4

    @functools.partial(
        pl.kernel,
        out_type=jax.ShapeDtypeStruct((EPAD, d), F32),
        mesh=_MESH,
        compiler_params=_SC_PARAMS,
        scratch_types=[pltpu.VMEM((TPW, 128), I32)]
        + [pltpu.VMEM((128, d), F32)] * nb
        + [pltpu.SemaphoreType.DMA] * (2 * nb),
    )
    def k(tab_hbm, idx_hbm, out_hbm, idx_v, *bs):
        bufs = bs[:nb]
        gsem = bs[nb:2 * nb]
        ssem = bs[2 * nb:]
        w = _wid()
        pltpu.sync_copy(idx_hbm.at[w], idx_v)
        for t in range(nb):
            pltpu.async_copy(tab_hbm.at[idx_v.at[t]], bufs[t], gsem[t])

        def step(j, carry):
            for t in range(nb):
                k0 = j * nb + t
                pltpu.make_async_copy(
                    tab_hbm.at[idx_v.at[0]], bufs[t], gsem[t]).wait()
                pltpu.async_copy(
                    bufs[t], out_hbm.at[pl.ds((w * TPW + k0) * 128, 128)],
                    ssem[t])
                kn = k0 + nb

                @pl.when(kn < TPW)
                def _():
                    pltpu.make_async_copy(
                        bufs[t], out_hbm.at[pl.ds(0, 128)], ssem[t]).wait()
                    pltpu.async_copy(
                        tab_hbm.at[idx_v.at[kn]], bufs[t], gsem[t])

            return carry

        lax.fori_loop(0, TPW // nb, step, 0)
        for t in range(nb):
            pltpu.make_async_copy(
                bufs[t], out_hbm.at[pl.ds(0, 128)], ssem[t]).wait()

    return k


def _make_sc_alpha(use_base, use_src):
    scratch = [pltpu.VMEM((NPAD,), F32)]
    if use_src:
        scratch.append(pltpu.VMEM((NPAD,), F32))
        scratch.append(pltpu.VMEM((GBLK,), I32))
    if use_base:
        scratch.append(pltpu.VMEM((GBLK,), F32))
    scratch += [
        pltpu.VMEM((GBLK,), I32),
        pltpu.VMEM((GBLK,), F32),
        pltpu.VMEM((16,), F32),
    ]

    @functools.partial(
        pl.kernel,
        out_type=(
            jax.ShapeDtypeStruct((EPAD,), F32),
            jax.ShapeDtypeStruct((32, 1, 16), F32),
        ),
        mesh=_MESH,
        compiler_params=_SC_PARAMS,
        scratch_types=scratch,
    )
    def k(*refs):
        it = iter(refs)
        base_hbm = next(it) if use_base else None
        src_hbm = next(it) if use_src else None
        dst_hbm = next(it)
        t1_hbm = next(it) if use_src else None
        t2_hbm = next(it)
        a_hbm = next(it)
        mx_hbm = next(it)
        t2_v = next(it)
        t1_v = next(it) if use_src else None
        s_v = next(it) if use_src else None
        base_v = next(it) if use_base else None
        d_v = next(it)
        st_v = next(it)
        mx_v = next(it)

        w = _wid()
        ebase = w * TPW * 128
        pltpu.sync_copy(t2_hbm, t2_v)
        if use_src:
            pltpu.sync_copy(t1_hbm, t1_v)

        mx = jnp.full((16,), -3.0e38, F32)
        for b in range(TPW * 128 // GBLK):
            off = ebase + b * GBLK
            if use_base:
                pltpu.sync_copy(base_hbm.at[pl.ds(off, GBLK)], base_v)
            if use_src:
                pltpu.sync_copy(src_hbm.at[pl.ds(off, GBLK)], s_v)
            pltpu.sync_copy(dst_hbm.at[pl.ds(off, GBLK)], d_v)

            def body(j, m):
                sl = pl.ds(j * 16, 16)
                v = plsc.load_gather(t2_v, [d_v[sl]])
                if use_src:
                    v = v + plsc.load_gather(t1_v, [s_v[sl]])
                if use_base:
                    v = v + base_v[sl]
                v = jnp.where(v > 0, v, 0.01 * v)
                st_v[sl] = v
                return jnp.maximum(m, v)

            mx = lax.fori_loop(0, GBLK // 16, body, mx)
            pltpu.sync_copy(st_v, a_hbm.at[pl.ds(off, GBLK)])
        mx_v[...] = mx
        pltpu.sync_copy(mx_v, mx_hbm.at[w, 0])

    return k


def _make_sc_exp_hist():
    @functools.partial(
        pl.kernel,
        out_type=(
            jax.ShapeDtypeStruct((EPAD,), F32),
            jax.ShapeDtypeStruct((32, 1, NPAD), F32),
        ),
        mesh=_MESH,
        compiler_params=_SC_PARAMS,
        scratch_types=[
            pltpu.VMEM((NPAD,), F32),
            pltpu.VMEM((GBLK,), F32),
            pltpu.VMEM((GBLK,), I32),
            pltpu.VMEM((GBLK,), F32),
            pltpu.VMEM((32, 1, 16), F32),
        ],
    )
    def k(a_hbm, dst_hbm, mx_hbm, e_hbm, part_hbm, den_v, a_v, d_v, e_v, tm_v):
        w = _wid()
        ebase = w * TPW * 128
        pltpu.sync_copy(mx_hbm, tm_v)
        m = tm_v[0, 0]
        for i in range(1, 32):
            m = jnp.maximum(m, tm_v[i, 0])
        g = jnp.max(m)

        def zbody(i, c):
            den_v[pl.ds(i * 16, 16)] = jnp.zeros((16,), F32)
            return c

        lax.fori_loop(0, NPAD // 16, zbody, 0)

        for b in range(TPW * 128 // GBLK):
            off = ebase + b * GBLK
            pltpu.sync_copy(a_hbm.at[pl.ds(off, GBLK)], a_v)
            pltpu.sync_copy(dst_hbm.at[pl.ds(off, GBLK)], d_v)

            def body(j, c):
                sl = pl.ds(j * 16, 16)
                e16 = jnp.exp(a_v[sl] - g)
                e_v[sl] = e16
                plsc.addupdate_scatter(den_v, [d_v[sl]], e16)
                return c

            lax.fori_loop(0, GBLK // 16, body, 0)
            pltpu.sync_copy(e_v, e_hbm.at[pl.ds(off, GBLK)])
        pltpu.sync_copy(den_v, part_hbm.at[w, 0])

    return k


def _make_sc_w():
    @functools.partial(
        pl.kernel,
        out_type=jax.ShapeDtypeStruct((EPAD,), F32),
        mesh=_MESH,
        compiler_params=_SC_PARAMS,
        scratch_types=[
            pltpu.VMEM((NPAD,), F32),
            pltpu.VMEM((GBLK,), F32),
            pltpu.VMEM((GBLK,), I32),
            pltpu.VMEM((GBLK,), F32),
        ],
    )
    def k(e_hbm, dst_hbm, den_hbm, w_hbm, den_v, e_v, d_v, w_v):
        w = _wid()
        ebase = w * TPW * 128
        pltpu.sync_copy(den_hbm, den_v)
        for b in range(TPW * 128 // GBLK):
            off = ebase + b * GBLK
            pltpu.sync_copy(e_hbm.at[pl.ds(off, GBLK)], e_v)
            pltpu.sync_copy(dst_hbm.at[pl.ds(off, GBLK)], d_v)

            def body(j, c):
                sl = pl.ds(j * 16, 16)
                den16 = plsc.load_gather(den_v, [d_v[sl]])
                w_v[sl] = e_v[sl] / (den16 + 1e-16)
                return c

            lax.fori_loop(0, GBLK // 16, body, 0)
            pltpu.sync_copy(w_v, w_hbm.at[pl.ds(off, GBLK)])

    return k


def _make_sc_deg():
    @functools.partial(
        pl.kernel,
        out_type=jax.ShapeDtypeStruct((32, 1, NPAD), F32),
        mesh=_MESH,
        compiler_params=_SC_PARAMS,
        scratch_types=[
            pltpu.VMEM((NPAD,), F32),
            pltpu.VMEM((GBLK,), I32),
        ],
    )
    def k(src_hbm, part_hbm, deg_v, s_v):
        w = _wid()
        ebase = w * TPW * 128

        def zbody(i, c):
            deg_v[pl.ds(i * 16, 16)] = jnp.zeros((16,), F32)
            return c

        lax.fori_loop(0, NPAD // 16, zbody, 0)
        ones = jnp.full((16,), 1.0, F32)
        for b in range(TPW * 128 // GBLK):
            off = ebase + b * GBLK
            pltpu.sync_copy(src_hbm.at[pl.ds(off, GBLK)], s_v)

            def body(j, c):
                plsc.addupdate_scatter(deg_v, [s_v[pl.ds(j * 16, 16)]], ones)
                return c

            lax.fori_loop(0, GBLK // 16, body, 0)
        pltpu.sync_copy(deg_v, part_hbm.at[w, 0])

    return k


def _make_sc_scatter():
    cpw = ROWS // 16

    @functools.partial(
        pl.kernel,
        out_type=jax.ShapeDtypeStruct((2 * HALFP, 128), F32),
        mesh=_MESH,
        compiler_params=_SC_PARAMS,
        scratch_types=[
            pltpu.VMEM_SHARED((ACCQ * 128, 128), F32),
            pltpu.VMEM((64, 128), F32),
            pltpu.VMEM((128,), I32),
            pltpu.VMEM((64,), I32),
            pltpu.VMEM((64,), I32),
        ],
    )
    def k(msg_hbm, dst_hbm, out_hbm, acc, rows_v, d_v, la_v, lb_v):
        c = lax.axis_index("c")
        s = lax.axis_index("s")

        def zv(i, cc):
            for q in range(8):
                rows_v[i, pl.ds(q * 16, 16)] = jnp.zeros((16,), F32)
            return cc

        lax.fori_loop(0, 64, zv, 0)

        def zacc(i, cc):
            ch = s + i * 16

            @pl.when(ch < 2 * ACCQ)
            def _():
                pltpu.sync_copy(rows_v, acc.at[pl.ds(ch * 64, 64)])

            return cc

        lax.fori_loop(0, (2 * ACCQ + 15) // 16, zacc, 0)
        plsc.subcore_barrier()

        pbase = c * HALFP

        def body(j, cc):
            chunk = s * cpw + j
            pltpu.sync_copy(dst_hbm.at[pl.ds(chunk * 128, 128)], d_v)
            for half, l_v in ((0, la_v), (1, lb_v)):
                for gi in range(4):
                    sl = pl.ds(half * 64 + gi * 16, 16)
                    loc = lax.shift_right_logical(d_v[sl], 1) - pbase
                    ok = jnp.logical_and(loc >= 0, loc < HALFP)
                    l_v[pl.ds(gi * 16, 16)] = jnp.where(ok, loc, HALFP)
            for half, l_v in ((0, la_v), (1, lb_v)):
                pltpu.sync_copy(
                    msg_hbm.at[pl.ds(chunk * 128 + half * 64, 64)], rows_v)
                pltpu.sync_copy(rows_v, acc.at[l_v], add=True)
            return cc

        lax.fori_loop(0, cpw, body, 0)
        plsc.subcore_barrier()
        rpw = HALFP // 16
        pltpu.sync_copy(
            acc.at[pl.ds(s * rpw, rpw)],
            out_hbm.at[pl.ds(c * HALFP + s * rpw, rpw)],
        )

    return k


_sc_gather128 = _make_sc_gather(128)
_sc_alpha_gat = _make_sc_alpha(use_base=False, use_src=True)
_sc_alpha_gate = _make_sc_alpha(use_base=True, use_src=False)
_sc_exp_hist = _make_sc_exp_hist()
_sc_w = _make_sc_w()
_sc_deg = _make_sc_deg()
_sc_scatter = _make_sc_scatter()


def _npad1(v):
    return jnp.pad(v, (0, NPAD - N_NODES))


def _npad2(m):
    return jnp.pad(m, ((0, NPAD - m.shape[0]), (0, 0)))


def _scatter_nodes(m, dstp):
    z = jnp.zeros_like(m)
    left = (dstp & 1) == 0
    msg = jnp.where(left[:, None], jnp.concatenate([m, z], axis=1),
                    jnp.concatenate([z, m], axis=1))
    out = _sc_scatter(msg, dstp).reshape(4 * HALFP, 64)
    return out[:N_NODES]


def _softmax_weights(alpha, tmax, dstp):
    e, parts = _sc_exp_hist(alpha, dstp, tmax)
    denom = parts.sum(axis=(0, 1))
    return _sc_w(e, dstp, denom)


_BLK = 1024


def _matmul_kernel(x_ref, w_ref, b_ref, o_ref):
    o_ref[...] = jnp.dot(x_ref[...], w_ref[...],
                         preferred_element_type=jnp.float32) + b_ref[...]


def _pallas_linear(x, w_t, b):
    n, kk = x.shape
    m = w_t.shape[1]
    npad = (-n) % _BLK
    xp = jnp.pad(x, ((0, npad), (0, 0)))
    grid = (xp.shape[0] // _BLK,)
    out = pl.pallas_call(
        _matmul_kernel,
        grid=grid,
        in_specs=[
            pl.BlockSpec((_BLK, kk), lambda i: (i, 0)),
            pl.BlockSpec((kk, m), lambda i: (0, 0)),
            pl.BlockSpec((1, m), lambda i: (0, 0)),
        ],
        out_specs=pl.BlockSpec((_BLK, m), lambda i: (i, 0)),
        out_shape=jax.ShapeDtypeStruct((xp.shape[0], m), jnp.float32),
    )(xp, w_t, b.reshape(1, m))
    return out[:n]


def _gate_conv_sc(x0, srcp, src2d, dstp, edge_attr, p):
    w1 = p['gate_lin1_w']
    u = x0 @ w1[:, :64].T
    y2 = x0 @ p['gate_lin2_w'].T
    rd = x0 @ p['gate_att_r']
    uy = _sc_gather128(_npad2(jnp.concatenate([u, y2], axis=1)), src2d)
    eproj = jnp.pad(edge_attr, ((0, EPAD - N_EDGES), (0, 0))) @ w1[:, 64:].T
    tdot = _leaky(uy[:, :64] + eproj) @ p['gate_att_l']
    alpha, tmax = _sc_alpha_gate(tdot, dstp, _npad1(rd))
    wts = _softmax_weights(alpha, tmax, dstp)
    return _scatter_nodes(uy[:, 64:] * wts[:, None], dstp) + p['gate_bias']


def _gat_conv_sc(xg, srcp, src2d, dstp, w, att_src, att_dst, bias):
    xp = xg @ w.T
    ssrc = xp @ att_src
    sdst = xp @ att_dst
    alpha, tmax = _sc_alpha_gat(srcp, dstp, _npad1(ssrc), _npad1(sdst))
    wts = _softmax_weights(alpha, tmax, dstp)
    xpj = _sc_gather128(_npad2(jnp.concatenate([xp, xp], axis=1)), src2d)
    return _scatter_nodes(xpj[:, 64:] * wts[:, None], dstp) + bias


def kernel(x, edge_attr, params, edge_index, batch):
    p = params
    src = edge_index[0].astype(I32)
    dst = edge_index[1].astype(I32)
    pad = jnp.full((EPAD - N_EDGES,), DUMMY, I32)
    srcp = jnp.concatenate([src, pad])
    dstp = jnp.concatenate([dst, pad])
    src2d = srcp.reshape(32, TPW, 128)

    deg_parts = _sc_deg(srcp)
    outdeg = deg_parts.sum(axis=(0, 1))[:N_NODES]
    has_edges = jax.ops.segment_sum(outdeg, batch, N_GRAPHS) > 0.5

    x0 = _leaky(_pallas_linear(x, p['lin1_w'].T, p['lin1_b']))
    h = _elu(_gate_conv_sc(x0, srcp, src2d, dstp, edge_attr, p))
    xg = jax.nn.relu(_gru(h, x0, p['gru0_wih'], p['gru0_whh'],
                          p['gru0_bih'], p['gru0_bhh']))
    for i in range(2):
        h = _elu(_gat_conv_sc(xg, srcp, src2d, dstp, p['conv%d_w' % i],
                              p['conv%d_att_src' % i], p['conv%d_att_dst' % i],
                              p['conv%d_bias' % i]))
        xg = jax.nn.relu(_gru(h, xg, p['agru%d_wih' % i], p['agru%d_whh' % i],
                              p['agru%d_bih' % i], p['agru%d_bhh' % i]))
    out = jax.nn.relu(jax.ops.segment_sum(xg, batch, N_GRAPHS))
    xs = xg @ p['mol_w'].T
    a_src = xs @ p['mol_att_src']
    for _ in range(2):
        xd = out @ p['mol_w'].T
        a = _leaky(a_src + (xd @ p['mol_att_dst'])[batch], 0.01)
        alpha = _segment_softmax(a, batch, N_GRAPHS)
        h = _elu(jax.ops.segment_sum(alpha[:, None] * xs, batch, N_GRAPHS)
                 + p['mol_bias'])
        out = jax.nn.relu(_gru(h, out, p['mgru_wih'], p['mgru_whh'],
                               p['mgru_bih'], p['mgru_bhh']))
    gnn_out = out
    xl = x0 @ p['linlone_w'].T + p['linlone_b']
    atom_out = jax.ops.segment_sum(xl, batch, N_GRAPHS)
    out = jnp.where(has_edges[:, None], gnn_out, atom_out)
    return _pallas_linear(out, p['lin2_w'].T, p['lin2_b'])


# speedup vs baseline: 6.5952x; 1.1351x over previous
import functools

import jax
import jax.numpy as jnp
from jax import lax
from jax.experimental import pallas as pl
from jax.experimental.pallas import tpu as pltpu
from jax.experimental.pallas import tpu_sc as plsc

N_GRAPHS = 512
N_NODES = 50000
N_EDGES = 800000

EPAD = 802816
ROWS = EPAD // 128
TPW = ROWS // 32
NPAD = 50176
DUMMY = 50000
HALFP = 12800
ACCQ = 101
GBLK = 1792
F32 = jnp.float32
I32 = jnp.int32

_MESH = plsc.VectorSubcoreMesh(core_axis_name="c", subcore_axis_name="s")
_SC_PARAMS = pltpu.CompilerParams(needs_layout_passes=False)


def _wid():
    return lax.axis_index("s") * 2 + lax.axis_index("c")


def _leaky(x, s=0.01):
    return jnp.where(x > 0, x, s * x)


def _elu(x):
    return jnp.where(x > 0, x, jnp.expm1(x))


def _gru(x, h, wih, whh, bih, bhh):
    gi = x @ wih.T + bih
    gh = h @ whh.T + bhh
    ir, iz, inn = jnp.split(gi, 3, axis=-1)
    hr, hz, hn = jnp.split(gh, 3, axis=-1)
    r = jax.nn.sigmoid(ir + hr)
    z = jax.nn.sigmoid(iz + hz)
    n = jnp.tanh(inn + r * hn)
    return (1.0 - z) * n + z * h


def _segment_softmax(alpha, index, num_segments):
    amax = jax.ops.segment_max(alpha, index, num_segments)
    amax = jnp.where(jnp.isfinite(amax), amax, 0.0)
    a = jnp.exp(alpha - amax[index])
    denom = jax.ops.segment_sum(a, index, num_segments)
    return a / (denom[index] + 1e-16)


def _make_sc_gather(d):
    nb = 4

    @functools.partial(
        pl.kernel,
        out_type=jax.ShapeDtypeStruct((EPAD, d), F32),
        mesh=_MESH,
        compiler_params=_SC_PARAMS,
        scratch_types=[pltpu.VMEM((TPW, 128), I32)]
        + [pltpu.VMEM((128, d), F32)] * nb
        + [pltpu.SemaphoreType.DMA] * (2 * nb),
    )
    def k(tab_hbm, idx_hbm, out_hbm, idx_v, *bs):
        bufs = bs[:nb]
        gsem = bs[nb:2 * nb]
        ssem = bs[2 * nb:]
        w = _wid()
        pltpu.sync_copy(idx_hbm.at[w], idx_v)
        for t in range(nb):
            pltpu.async_copy(tab_hbm.at[idx_v.at[t]], bufs[t], gsem[t])

        def step(j, carry):
            for t in range(nb):
                k0 = j * nb + t
                pltpu.make_async_copy(
                    tab_hbm.at[idx_v.at[0]], bufs[t], gsem[t]).wait()
                pltpu.async_copy(
                    bufs[t], out_hbm.at[pl.ds((w * TPW + k0) * 128, 128)],
                    ssem[t])
                kn = k0 + nb

                @pl.when(kn < TPW)
                def _():
                    pltpu.make_async_copy(
                        bufs[t], out_hbm.at[pl.ds(0, 128)], ssem[t]).wait()
                    pltpu.async_copy(
                        tab_hbm.at[idx_v.at[kn]], bufs[t], gsem[t])

            return carry

        lax.fori_loop(0, TPW // nb, step, 0)
        for t in range(nb):
            pltpu.make_async_copy(
                bufs[t], out_hbm.at[pl.ds(0, 128)], ssem[t]).wait()

    return k


def _make_sc_alpha(use_base, use_src):
    scratch = [pltpu.VMEM((NPAD,), F32)]
    if use_src:
        scratch.append(pltpu.VMEM((NPAD,), F32))
        scratch.append(pltpu.VMEM((GBLK,), I32))
    if use_base:
        scratch.append(pltpu.VMEM((GBLK,), F32))
    scratch += [
        pltpu.VMEM((GBLK,), I32),
        pltpu.VMEM((GBLK,), F32),
        pltpu.VMEM((16,), F32),
    ]

    @functools.partial(
        pl.kernel,
        out_type=(
            jax.ShapeDtypeStruct((EPAD,), F32),
            jax.ShapeDtypeStruct((32, 1, 16), F32),
        ),
        mesh=_MESH,
        compiler_params=_SC_PARAMS,
        scratch_types=scratch,
    )
    def k(*refs):
        it = iter(refs)
        base_hbm = next(it) if use_base else None
        src_hbm = next(it) if use_src else None
        dst_hbm = next(it)
        t1_hbm = next(it) if use_src else None
        t2_hbm = next(it)
        a_hbm = next(it)
        mx_hbm = next(it)
        t2_v = next(it)
        t1_v = next(it) if use_src else None
        s_v = next(it) if use_src else None
        base_v = next(it) if use_base else None
        d_v = next(it)
        st_v = next(it)
        mx_v = next(it)

        w = _wid()
        ebase = w * TPW * 128
        pltpu.sync_copy(t2_hbm, t2_v)
        if use_src:
            pltpu.sync_copy(t1_hbm, t1_v)

        mx = jnp.full((16,), -3.0e38, F32)
        for b in range(TPW * 128 // GBLK):
            off = ebase + b * GBLK
            if use_base:
                pltpu.sync_copy(base_hbm.at[pl.ds(off, GBLK)], base_v)
            if use_src:
                pltpu.sync_copy(src_hbm.at[pl.ds(off, GBLK)], s_v)
            pltpu.sync_copy(dst_hbm.at[pl.ds(off, GBLK)], d_v)

            def body(j, m):
                sl = pl.ds(j * 16, 16)
                v = plsc.load_gather(t2_v, [d_v[sl]])
                if use_src:
                    v = v + plsc.load_gather(t1_v, [s_v[sl]])
                if use_base:
                    v = v + base_v[sl]
                v = jnp.where(v > 0, v, 0.01 * v)
                st_v[sl] = v
                return jnp.maximum(m, v)

            mx = lax.fori_loop(0, GBLK // 16, body, mx)
            pltpu.sync_copy(st_v, a_hbm.at[pl.ds(off, GBLK)])
        mx_v[...] = mx
        pltpu.sync_copy(mx_v, mx_hbm.at[w, 0])

    return k


def _make_sc_exp_hist():
    @functools.partial(
        pl.kernel,
        out_type=(
            jax.ShapeDtypeStruct((EPAD,), F32),
            jax.ShapeDtypeStruct((32, 1, NPAD), F32),
        ),
        mesh=_MESH,
        compiler_params=_SC_PARAMS,
        scratch_types=[
            pltpu.VMEM((NPAD,), F32),
            pltpu.VMEM((GBLK,), F32),
            pltpu.VMEM((GBLK,), I32),
            pltpu.VMEM((GBLK,), F32),
            pltpu.VMEM((32, 1, 16), F32),
        ],
    )
    def k(a_hbm, dst_hbm, mx_hbm, e_hbm, part_hbm, den_v, a_v, d_v, e_v, tm_v):
        w = _wid()
        ebase = w * TPW * 128
        pltpu.sync_copy(mx_hbm, tm_v)
        m = tm_v[0, 0]
        for i in range(1, 32):
            m = jnp.maximum(m, tm_v[i, 0])
        g = jnp.max(m)

        def zbody(i, c):
            den_v[pl.ds(i * 16, 16)] = jnp.zeros((16,), F32)
            return c

        lax.fori_loop(0, NPAD // 16, zbody, 0)

        for b in range(TPW * 128 // GBLK):
            off = ebase + b * GBLK
            pltpu.sync_copy(a_hbm.at[pl.ds(off, GBLK)], a_v)
            pltpu.sync_copy(dst_hbm.at[pl.ds(off, GBLK)], d_v)

            def body(j, c):
                sl = pl.ds(j * 16, 16)
                e16 = jnp.exp(a_v[sl] - g)
                e_v[sl] = e16
                plsc.addupdate_scatter(den_v, [d_v[sl]], e16)
                return c

            lax.fori_loop(0, GBLK // 16, body, 0)
            pltpu.sync_copy(e_v, e_hbm.at[pl.ds(off, GBLK)])
        pltpu.sync_copy(den_v, part_hbm.at[w, 0])

    return k


def _make_sc_w():
    @functools.partial(
        pl.kernel,
        out_type=jax.ShapeDtypeStruct((EPAD,), F32),
        mesh=_MESH,
        compiler_params=_SC_PARAMS,
        scratch_types=[
            pltpu.VMEM((NPAD,), F32),
            pltpu.VMEM((GBLK,), F32),
            pltpu.VMEM((GBLK,), I32),
            pltpu.VMEM((GBLK,), F32),
        ],
    )
    def k(e_hbm, dst_hbm, den_hbm, w_hbm, den_v, e_v, d_v, w_v):
        w = _wid()
        ebase = w * TPW * 128
        pltpu.sync_copy(den_hbm, den_v)
        for b in range(TPW * 128 // GBLK):
            off = ebase + b * GBLK
            pltpu.sync_copy(e_hbm.at[pl.ds(off, GBLK)], e_v)
            pltpu.sync_copy(dst_hbm.at[pl.ds(off, GBLK)], d_v)

            def body(j, c):
                sl = pl.ds(j * 16, 16)
                den16 = plsc.load_gather(den_v, [d_v[sl]])
                w_v[sl] = e_v[sl] / (den16 + 1e-16)
                return c

            lax.fori_loop(0, GBLK // 16, body, 0)
            pltpu.sync_copy(w_v, w_hbm.at[pl.ds(off, GBLK)])

    return k


def _make_sc_deg():
    @functools.partial(
        pl.kernel,
        out_type=jax.ShapeDtypeStruct((32, 1, NPAD), F32),
        mesh=_MESH,
        compiler_params=_SC_PARAMS,
        scratch_types=[
            pltpu.VMEM((NPAD,), F32),
            pltpu.VMEM((GBLK,), I32),
        ],
    )
    def k(src_hbm, part_hbm, deg_v, s_v):
        w = _wid()
        ebase = w * TPW * 128

        def zbody(i, c):
            deg_v[pl.ds(i * 16, 16)] = jnp.zeros((16,), F32)
            return c

        lax.fori_loop(0, NPAD // 16, zbody, 0)
        ones = jnp.full((16,), 1.0, F32)
        for b in range(TPW * 128 // GBLK):
            off = ebase + b * GBLK
            pltpu.sync_copy(src_hbm.at[pl.ds(off, GBLK)], s_v)

            def body(j, c):
                plsc.addupdate_scatter(deg_v, [s_v[pl.ds(j * 16, 16)]], ones)
                return c

            lax.fori_loop(0, GBLK // 16, body, 0)
        pltpu.sync_copy(deg_v, part_hbm.at[w, 0])

    return k


def _make_sc_scatter():
    cpw = ROWS // 16

    @functools.partial(
        pl.kernel,
        out_type=jax.ShapeDtypeStruct((2 * HALFP, 128), F32),
        mesh=_MESH,
        compiler_params=_SC_PARAMS,
        scratch_types=[
            pltpu.VMEM_SHARED((ACCQ * 128, 128), F32),
            pltpu.VMEM((128,), I32),
            pltpu.VMEM((64, 128), F32),
            pltpu.VMEM((64, 128), F32),
            pltpu.VMEM((64,), I32),
            pltpu.VMEM((64,), I32),
            pltpu.SemaphoreType.DMA,
            pltpu.SemaphoreType.DMA,
            pltpu.SemaphoreType.DMA,
            pltpu.SemaphoreType.DMA,
        ],
    )
    def k(msg_hbm, dst_hbm, out_hbm, acc, d_v, ma_v, mb_v, la_v, lb_v,
          msa, msb, asa, asb):
        c = lax.axis_index("c")
        s = lax.axis_index("s")

        def zv(i, cc):
            for q in range(8):
                ma_v[i, pl.ds(q * 16, 16)] = jnp.zeros((16,), F32)
            return cc

        lax.fori_loop(0, 64, zv, 0)

        def zacc(i, cc):
            ch = s + i * 16

            @pl.when(ch < 2 * ACCQ)
            def _():
                pltpu.sync_copy(ma_v, acc.at[pl.ds(ch * 64, 64)])

            return cc

        lax.fori_loop(0, (2 * ACCQ + 15) // 16, zacc, 0)
        plsc.subcore_barrier()

        pbase = c * HALFP
        slots = ((0, ma_v, la_v, msa, asa), (1, mb_v, lb_v, msb, asb))

        def body(j, cc):
            chunk = s * cpw + j
            pltpu.sync_copy(dst_hbm.at[pl.ds(chunk * 128, 128)], d_v)
            for half, m_v, l_v, msem, asem in slots:

                @pl.when(j > 0)
                def _():
                    pltpu.make_async_copy(m_v, acc.at[l_v], asem).wait()

                pltpu.async_copy(
                    msg_hbm.at[pl.ds(chunk * 128 + half * 64, 64)],
                    m_v, msem)
            for half, m_v, l_v, msem, asem in slots:
                for gi in range(4):
                    sl = pl.ds(half * 64 + gi * 16, 16)
                    loc = lax.shift_right_logical(d_v[sl], 1) - pbase
                    ok = jnp.logical_and(loc >= 0, loc < HALFP)
                    l_v[pl.ds(gi * 16, 16)] = jnp.where(ok, loc, HALFP)
                pltpu.make_async_copy(
                    msg_hbm.at[pl.ds(0, 64)], m_v, msem).wait()
                pltpu.async_copy(m_v, acc.at[l_v], asem, add=True)
            return cc

        lax.fori_loop(0, cpw, body, 0)
        for _, m_v, l_v, msem, asem in slots:
            pltpu.make_async_copy(m_v, acc.at[l_v], asem).wait()
        plsc.subcore_barrier()
        rpw = HALFP // 16
        pltpu.sync_copy(
            acc.at[pl.ds(s * rpw, rpw)],
            out_hbm.at[pl.ds(c * HALFP + s * rpw, rpw)],
        )

    return k


_sc_gather128 = _make_sc_gather(128)
_sc_alpha_gat = _make_sc_alpha(use_base=False, use_src=True)
_sc_alpha_gate = _make_sc_alpha(use_base=True, use_src=False)
_sc_exp_hist = _make_sc_exp_hist()
_sc_w = _make_sc_w()
_sc_deg = _make_sc_deg()
_sc_scatter = _make_sc_scatter()


def _npad1(v):
    return jnp.pad(v, (0, NPAD - N_NODES))


def _npad2(m):
    return jnp.pad(m, ((0, NPAD - m.shape[0]), (0, 0)))


def _scatter_nodes(m, dstp):
    z = jnp.zeros_like(m)
    left = (dstp & 1) == 0
    msg = jnp.where(left[:, None], jnp.concatenate([m, z], axis=1),
                    jnp.concatenate([z, m], axis=1))
    out = _sc_scatter(msg, dstp).reshape(4 * HALFP, 64)
    return out[:N_NODES]


def _softmax_weights(alpha, tmax, dstp):
    e, parts = _sc_exp_hist(alpha, dstp, tmax)
    denom = parts.sum(axis=(0, 1))
    return _sc_w(e, dstp, denom)


_BLK = 1024


def _matmul_kernel(x_ref, w_ref, b_ref, o_ref):
    o_ref[...] = jnp.dot(x_ref[...], w_ref[...],
                         preferred_element_type=jnp.float32) + b_ref[...]


def _pallas_linear(x, w_t, b):
    n, kk = x.shape
    m = w_t.shape[1]
    npad = (-n) % _BLK
    xp = jnp.pad(x, ((0, npad), (0, 0)))
    grid = (xp.shape[0] // _BLK,)
    out = pl.pallas_call(
        _matmul_kernel,
        grid=grid,
        in_specs=[
            pl.BlockSpec((_BLK, kk), lambda i: (i, 0)),
            pl.BlockSpec((kk, m), lambda i: (0, 0)),
            pl.BlockSpec((1, m), lambda i: (0, 0)),
        ],
        out_specs=pl.BlockSpec((_BLK, m), lambda i: (i, 0)),
        out_shape=jax.ShapeDtypeStruct((xp.shape[0], m), jnp.float32),
    )(xp, w_t, b.reshape(1, m))
    return out[:n]


def _gate_conv_sc(x0, srcp, src2d, dstp, edge_attr, p):
    w1 = p['gate_lin1_w']
    u = x0 @ w1[:, :64].T
    y2 = x0 @ p['gate_lin2_w'].T
    rd = x0 @ p['gate_att_r']
    uy = _sc_gather128(_npad2(jnp.concatenate([u, y2], axis=1)), src2d)
    eproj = jnp.pad(edge_attr, ((0, EPAD - N_EDGES), (0, 0))) @ w1[:, 64:].T
    tdot = _leaky(uy[:, :64] + eproj) @ p['gate_att_l']
    alpha, tmax = _sc_alpha_gate(tdot, dstp, _npad1(rd))
    wts = _softmax_weights(alpha, tmax, dstp)
    return _scatter_nodes(uy[:, 64:] * wts[:, None], dstp) + p['gate_bias']


def _gat_conv_sc(xg, srcp, src2d, dstp, w, att_src, att_dst, bias):
    xp = xg @ w.T
    ssrc = xp @ att_src
    sdst = xp @ att_dst
    alpha, tmax = _sc_alpha_gat(srcp, dstp, _npad1(ssrc), _npad1(sdst))
    wts = _softmax_weights(alpha, tmax, dstp)
    xpj = _sc_gather128(_npad2(jnp.concatenate([xp, xp], axis=1)), src2d)
    return _scatter_nodes(xpj[:, 64:] * wts[:, None], dstp) + bias


def kernel(x, edge_attr, params, edge_index, batch):
    p = params
    src = edge_index[0].astype(I32)
    dst = edge_index[1].astype(I32)
    pad = jnp.full((EPAD - N_EDGES,), DUMMY, I32)
    srcp = jnp.concatenate([src, pad])
    dstp = jnp.concatenate([dst, pad])
    src2d = srcp.reshape(32, TPW, 128)

    deg_parts = _sc_deg(srcp)
    outdeg = deg_parts.sum(axis=(0, 1))[:N_NODES]
    has_edges = jax.ops.segment_sum(outdeg, batch, N_GRAPHS) > 0.5

    x0 = _leaky(_pallas_linear(x, p['lin1_w'].T, p['lin1_b']))
    h = _elu(_gate_conv_sc(x0, srcp, src2d, dstp, edge_attr, p))
    xg = jax.nn.relu(_gru(h, x0, p['gru0_wih'], p['gru0_whh'],
                          p['gru0_bih'], p['gru0_bhh']))
    for i in range(2):
        h = _elu(_gat_conv_sc(xg, srcp, src2d, dstp, p['conv%d_w' % i],
                              p['conv%d_att_src' % i], p['conv%d_att_dst' % i],
                              p['conv%d_bias' % i]))
        xg = jax.nn.relu(_gru(h, xg, p['agru%d_wih' % i], p['agru%d_whh' % i],
                              p['agru%d_bih' % i], p['agru%d_bhh' % i]))
    out = jax.nn.relu(jax.ops.segment_sum(xg, batch, N_GRAPHS))
    xs = xg @ p['mol_w'].T
    a_src = xs @ p['mol_att_src']
    for _ in range(2):
        xd = out @ p['mol_w'].T
        a = _leaky(a_src + (xd @ p['mol_att_dst'])[batch], 0.01)
        alpha = _segment_softmax(a, batch, N_GRAPHS)
        h = _elu(jax.ops.segment_sum(alpha[:, None] * xs, batch, N_GRAPHS)
                 + p['mol_bias'])
        out = jax.nn.relu(_gru(h, out, p['mgru_wih'], p['mgru_whh'],
                               p['mgru_bih'], p['mgru_bhh']))
    gnn_out = out
    xl = x0 @ p['linlone_w'].T + p['linlone_b']
    atom_out = jax.ops.segment_sum(xl, batch, N_GRAPHS)
    out = jnp.where(has_edges[:, None], gnn_out, atom_out)
    return _pallas_linear(out, p['lin2_w'].T, p['lin2_b'])
